# Initial kernel scaffold; baseline (speedup 1.0000x reference)
import jax
import jax.numpy as jnp
from jax.experimental import pallas as pl


def kernel(input, C):
    raise NotImplementedError("write your pallas kernel here")


# speedup vs baseline: 6.2331x; 6.2331x over previous
import functools

import jax
import jax.numpy as jnp
from jax import lax
from jax.experimental import pallas as pl
from jax.experimental.pallas import tpu as pltpu
from jax.experimental.pallas import tpu_sc as plsc

CH = 128
K = 4
GROUP = K * CH


@functools.lru_cache(maxsize=None)
def _build(n_total: int, d: int):
  info = plsc.get_sparse_core_info()
  nc, ns = info.num_cores, info.num_subcores
  nw = nc * ns
  per_w = n_total // nw
  assert per_w * nw == n_total
  nch = per_w // CH
  assert nch * CH == per_w
  ng = nch // K
  assert ng * K == nch
  pairs = ng // 2
  assert pairs * 2 == ng and pairs >= 3

  mesh = plsc.VectorSubcoreMesh(core_axis_name="c", subcore_axis_name="s")

  @functools.partial(
      pl.kernel,
      out_type=jax.ShapeDtypeStruct((n_total, d), jnp.float32),
      mesh=mesh,
      compiler_params=pltpu.CompilerParams(use_tc_tiling_on_sc=False),
      scratch_types=[
          pltpu.VMEM((nch, CH), jnp.int32),
          pltpu.VMEM((2, GROUP, d), jnp.float32),
          pltpu.SemaphoreType.DMA,
          pltpu.SemaphoreType.DMA,
          pltpu.SemaphoreType.DMA,
          pltpu.SemaphoreType.DMA,
      ],
  )
  def gather_kernel(idx_hbm, tab_hbm, out_hbm, idx_v, rows, ga, gb, wa, wb):
    wid = lax.axis_index("s") * nc + lax.axis_index("c")
    base = wid * per_w
    pltpu.sync_copy(idx_hbm.at[wid], idx_v)

    def fire_gathers(g, s, sem):
      for kk in range(K):
        pltpu.async_copy(
            tab_hbm.at[idx_v.at[g * K + kk]],
            rows.at[s, pl.ds(kk * CH, CH)],
            sem)

    def wait_gathers(s, sem):
      for kk in range(K):
        pltpu.make_async_copy(
            tab_hbm.at[idx_v.at[kk]],
            rows.at[s, pl.ds(kk * CH, CH)],
            sem).wait()

    def fire_write(g, s, sem):
      pltpu.async_copy(
          rows.at[s], out_hbm.at[pl.ds(base + g * GROUP, GROUP)], sem)

    def wait_write(s, sem):
      pltpu.make_async_copy(
          rows.at[s], out_hbm.at[pl.ds(base, GROUP)], sem).wait()

    fire_gathers(0, 0, ga)
    fire_gathers(1, 1, gb)
    wait_gathers(0, ga)
    fire_write(0, 0, wa)
    wait_write(0, wa)
    fire_gathers(2, 0, ga)
    wait_gathers(1, gb)
    fire_write(1, 1, wb)

    def pair_body(p, carry):
      g0 = 2 * p
      wait_write(1, wb)
      fire_gathers(g0 + 1, 1, gb)
      wait_gathers(0, ga)
      fire_write(g0, 0, wa)
      wait_write(0, wa)
      fire_gathers(g0 + 2, 0, ga)
      wait_gathers(1, gb)
      fire_write(g0 + 1, 1, wb)
      return carry

    lax.fori_loop(1, pairs - 1, pair_body, 0)

    g0 = 2 * (pairs - 1)
    wait_write(1, wb)
    fire_gathers(g0 + 1, 1, gb)
    wait_gathers(0, ga)
    fire_write(g0, 0, wa)
    wait_gathers(1, gb)
    fire_write(g0 + 1, 1, wb)
    wait_write(0, wa)
    wait_write(1, wb)

  return gather_kernel, nw, nch


def kernel(input, C):
  idx = input.astype(jnp.int32)
  n_total = idx.size
  d = C.shape[1]
  gather_kernel, nw, nch = _build(n_total, d)
  idx3 = idx.reshape(nw, nch, CH)
  out = gather_kernel(idx3, C)
  return out.reshape(input.shape + (d,))
